# speedup vs baseline: 1.0774x; 1.0774x over previous
import functools
import math

import jax
import jax.numpy as jnp
from jax import lax
from jax.experimental import pallas as pl
from jax.experimental.pallas import tpu as pltpu

N_NODES = 10000
N_GRAPHS = 64
BN_SCALE = 1.0 / math.sqrt(1.0 + 1e-5)

ROW_BLK = 400


def _tc_proj_body(x_ref, w_ref, att_ref, h_ref, a_ref, *, heads, dout):
    h = jnp.dot(x_ref[...], w_ref[...], preferred_element_type=jnp.float32)
    h_ref[...] = h
    cols = []
    for which in range(2):
        for hd in range(4):
            if hd < heads:
                blk = h[:, hd * dout:(hd + 1) * dout]
                att = att_ref[4 * which + hd, :]
                cols.append(jnp.sum(blk * att[None, :], axis=1, keepdims=True))
            else:
                cols.append(jnp.zeros((h.shape[0], 1), jnp.float32))
    a_ref[...] = jnp.concatenate(cols, axis=1)


def _tc_proj(x, W, att_src, att_dst, heads, dout):
    n, din = x.shape
    hw = W.shape[1]
    att = jnp.zeros((8, dout), jnp.float32)
    att = att.at[0:heads, :].set(att_src)
    att = att.at[4:4 + heads, :].set(att_dst)
    grid = n // ROW_BLK
    h, a = pl.pallas_call(
        functools.partial(_tc_proj_body, heads=heads, dout=dout),
        grid=(grid,),
        in_specs=[
            pl.BlockSpec((ROW_BLK, din), lambda i: (i, 0)),
            pl.BlockSpec((din, hw), lambda i: (0, 0)),
            pl.BlockSpec((8, dout), lambda i: (0, 0)),
        ],
        out_specs=[
            pl.BlockSpec((ROW_BLK, hw), lambda i: (i, 0)),
            pl.BlockSpec((ROW_BLK, 8), lambda i: (i, 0)),
        ],
        out_shape=[
            jax.ShapeDtypeStruct((n, hw), jnp.float32),
            jax.ShapeDtypeStruct((n, 8), jnp.float32),
        ],
    )(x, W, att)
    return h, a


def _tc_mid_body(agg_ref, vecs_ref, w_ref, att_ref, h_ref, a_ref, *, heads, dout):
    bias = vecs_ref[0, :]
    gamma = vecs_ref[1, :]
    beta = vecs_ref[2, :]
    x2 = (agg_ref[...] + bias[None, :]) * (BN_SCALE * gamma)[None, :] + beta[None, :]
    x2 = jnp.maximum(x2, 0.0)
    h = jnp.dot(x2, w_ref[...], preferred_element_type=jnp.float32)
    h_ref[...] = h
    cols = []
    for which in range(2):
        for hd in range(4):
            if hd < heads:
                blk = h[:, hd * dout:(hd + 1) * dout]
                att = att_ref[4 * which + hd, :]
                cols.append(jnp.sum(blk * att[None, :], axis=1, keepdims=True))
            else:
                cols.append(jnp.zeros((h.shape[0], 1), jnp.float32))
    a_ref[...] = jnp.concatenate(cols, axis=1)


def _tc_mid(agg, bias, gamma, beta, W, att_src, att_dst, heads, dout):
    n, din = agg.shape
    hw = W.shape[1]
    att = jnp.zeros((8, dout), jnp.float32)
    att = att.at[0:heads, :].set(att_src)
    att = att.at[4:4 + heads, :].set(att_dst)
    vecs = jnp.stack([bias, gamma, beta], axis=0)
    grid = n // ROW_BLK
    h, a = pl.pallas_call(
        functools.partial(_tc_mid_body, heads=heads, dout=dout),
        grid=(grid,),
        in_specs=[
            pl.BlockSpec((ROW_BLK, din), lambda i: (i, 0)),
            pl.BlockSpec((3, din), lambda i: (0, 0)),
            pl.BlockSpec((din, hw), lambda i: (0, 0)),
            pl.BlockSpec((8, dout), lambda i: (0, 0)),
        ],
        out_specs=[
            pl.BlockSpec((ROW_BLK, hw), lambda i: (i, 0)),
            pl.BlockSpec((ROW_BLK, 8), lambda i: (i, 0)),
        ],
        out_shape=[
            jax.ShapeDtypeStruct((n, hw), jnp.float32),
            jax.ShapeDtypeStruct((n, 8), jnp.float32),
        ],
    )(agg, vecs, W, att)
    return h, a


def _tc_final_body(agg_ref, vecs_ref, batch_ref, wc_ref, out_ref, pooled_ref):
    i = pl.program_id(0)
    bias = vecs_ref[0, :]
    gamma = vecs_ref[1, :]
    beta = vecs_ref[2, :]
    x3 = (agg_ref[...] + bias[None, :]) * (BN_SCALE * gamma)[None, :] + beta[None, :]
    x3 = jnp.maximum(x3, 0.0)
    b = batch_ref[0, 0, :]
    gids = lax.broadcasted_iota(jnp.int32, (N_GRAPHS, ROW_BLK), 0)
    onehot = (gids == b[None, :]).astype(jnp.float32)

    @pl.when(i == 0)
    def _():
        pooled_ref[...] = jnp.zeros_like(pooled_ref)

    pooled_ref[...] += jnp.dot(onehot, x3, preferred_element_type=jnp.float32)

    @pl.when(i == pl.num_programs(0) - 1)
    def _():
        out_ref[...] = jnp.dot(pooled_ref[...], wc_ref[...],
                               preferred_element_type=jnp.float32)


def _tc_final(agg, bias, gamma, beta, batch, clf_W, clf_b):
    n, d = agg.shape
    vecs = jnp.stack([bias, gamma, beta], axis=0)
    grid = n // ROW_BLK
    batch3 = batch.reshape(grid, 1, ROW_BLK)
    wc_pad = jnp.zeros((d, 128), jnp.float32).at[:, :clf_W.shape[0]].set(clf_W.T)
    out = pl.pallas_call(
        _tc_final_body,
        grid=(grid,),
        in_specs=[
            pl.BlockSpec((ROW_BLK, d), lambda i: (i, 0)),
            pl.BlockSpec((3, d), lambda i: (0, 0)),
            pl.BlockSpec((1, 1, ROW_BLK), lambda i: (i, 0, 0)),
            pl.BlockSpec((d, 128), lambda i: (0, 0)),
        ],
        out_specs=pl.BlockSpec((N_GRAPHS, 128), lambda i: (0, 0)),
        out_shape=jax.ShapeDtypeStruct((N_GRAPHS, 128), jnp.float32),
        scratch_shapes=[pltpu.VMEM((N_GRAPHS, d), jnp.float32)],
    )(agg, vecs, batch3, wc_pad)
    return out[:, :clf_W.shape[0]] + clf_b[None, :]


def _edge_phase_jnp(h, a, src, dst, heads, dout):
    n = h.shape[0]
    a_src = a[:, 0:heads]
    a_dst = a[:, 4:4 + heads]
    alpha = a_src[src] + a_dst[dst]
    alpha = jax.nn.leaky_relu(alpha, 0.2)
    e = jnp.exp(alpha)
    denom = jax.ops.segment_sum(e, dst, num_segments=n)
    coef = e / (denom[dst] + 1e-16)
    hh = h.reshape(n, heads, dout)
    out = jax.ops.segment_sum(hh[src] * coef[:, :, None], dst, num_segments=n)
    return out.reshape(n, heads * dout)


def kernel(x, edge_index, batch, params):
    n = x.shape[0]
    loop = jnp.arange(n, dtype=edge_index.dtype)
    src = jnp.concatenate([edge_index[0], loop])
    dst = jnp.concatenate([edge_index[1], loop])

    p0, p1 = params['layers'][0], params['layers'][1]

    h1, a1 = _tc_proj(x, p0['W'], p0['att_src'], p0['att_dst'], 4, 256)
    agg1 = _edge_phase_jnp(h1, a1, src, dst, 4, 256)

    h2, a2 = _tc_mid(agg1, p0['bias'], p0['gamma'], p0['beta'],
                     p1['W'], p1['att_src'], p1['att_dst'], 1, 256)
    agg2 = _edge_phase_jnp(h2, a2, src, dst, 1, 256)

    return _tc_final(agg2, p1['bias'], p1['gamma'], p1['beta'],
                     batch, params['clf_W'], params['clf_b'])
